# Initial kernel scaffold; baseline (speedup 1.0000x reference)
import jax
import jax.numpy as jnp
from jax.experimental import pallas as pl


def kernel(x, W, edge_index, fc_w, fc_b, bn_gamma, bn_beta):
    raise NotImplementedError("write your pallas kernel here")


# speedup vs baseline: 1.7917x; 1.7917x over previous
import jax
import jax.numpy as jnp
from jax import lax
from jax.experimental import pallas as pl
from jax.experimental.pallas import tpu as pltpu
from jax.experimental.pallas import tpu_sc as plsc

N = 10000
E = 320000
D = 128
OUT = 128
J = 2

NC = 2
NS = 16
NW = NC * NS
EPW = E // NW
CH = 40
NCH = EPW // CH
N_PAD = 10240
RPS = N_PAD // NS

MM_BLOCK = 1000
BN_BLOCK = 1000


def _mm_body(x_ref, w_ref, y_ref):
    y_ref[...] = jnp.dot(x_ref[...], w_ref[...],
                         preferred_element_type=jnp.float32)


def _edge_body(y_hbm, src_hbm, dst_hbm, w_hbm, out_hbm,
               src_c, dst_c, wc_v, rows_v, msg_v, acc_sh, sem):
    c = lax.axis_index("c")
    s = lax.axis_index("s")
    wid = s * NC + c

    zeros16 = jnp.zeros((16,), jnp.float32)

    def _zrow(r, carry):
        for j in range(OUT // 16):
            msg_v[r, pl.ds(16 * j, 16)] = zeros16
        return carry
    lax.fori_loop(0, CH, _zrow, 0)
    for b in range(RPS // CH):
        pltpu.sync_copy(msg_v, acc_sh.at[pl.ds(s * RPS + b * CH, CH), :])
    plsc.subcore_barrier()

    def _chunk(ci, carry):
        pltpu.sync_copy(src_hbm.at[wid, ci], src_c)
        pltpu.sync_copy(dst_hbm.at[wid, ci], dst_c)
        pltpu.sync_copy(w_hbm.at[wid, ci], wc_v)

        pltpu.async_copy(y_hbm.at[src_c], rows_v, sem).wait()

        def _edge(k, kcarry):
            w0 = wc_v[k]
            w1 = wc_v[CH + k]
            for j in range(OUT // 16):
                a = rows_v[k, pl.ds(16 * j, 16)]
                b = rows_v[k, pl.ds(OUT + 16 * j, 16)]
                msg_v[k, pl.ds(16 * j, 16)] = w0 * a + w1 * b
            return kcarry
        lax.fori_loop(0, CH, _edge, 0)

        pltpu.sync_copy(msg_v, acc_sh.at[dst_c], add=True)
        return carry
    lax.fori_loop(0, NCH, _chunk, 0)

    plsc.subcore_barrier()
    pltpu.sync_copy(acc_sh.at[pl.ds(s * RPS, RPS), :],
                    out_hbm.at[c, pl.ds(s * RPS, RPS), :])


def _bn_stats_body(p_ref, b_ref, lin_ref, st_ref, acc_ref):
    i = pl.program_id(0)
    sm = jnp.sum(p_ref[...], axis=0) + b_ref[0]
    lin_ref[...] = sm
    blk = jnp.stack([jnp.sum(sm, axis=0), jnp.sum(sm * sm, axis=0)])

    @pl.when(i == 0)
    def _():
        acc_ref[...] = blk

    @pl.when(i > 0)
    def _():
        acc_ref[...] = acc_ref[...] + blk

    @pl.when(i == pl.num_programs(0) - 1)
    def _():
        st_ref[...] = acc_ref[...]


def _bn_norm_body(lin_ref, st_ref, g_ref, bb_ref, o_ref):
    inv_n = jnp.float32(1.0 / N)
    mean = st_ref[0] * inv_n
    var = st_ref[1] * inv_n - mean * mean
    scale = lax.rsqrt(var + jnp.float32(1e-5)) * g_ref[0]
    o_ref[...] = (lin_ref[...] - mean) * scale + bb_ref[0]


def kernel(x, W, edge_index, fc_w, fc_b, bn_gamma, bn_beta):
    wm = fc_w.reshape(OUT, J, D).transpose(2, 1, 0).reshape(D, J * OUT)
    src_r = edge_index[0].reshape(NW, NCH, CH)
    dst_r = edge_index[1].reshape(NW, NCH, CH)
    w_r = jnp.broadcast_to(
        W.reshape(NW, NCH, CH, J).transpose(0, 1, 3, 2)
         .reshape(NW, NCH, J * CH, 1),
        (NW, NCH, J * CH, 16)).astype(jnp.float32)

    y = pl.pallas_call(
        _mm_body,
        grid=(N // MM_BLOCK,),
        in_specs=[
            pl.BlockSpec((MM_BLOCK, D), lambda i: (i, 0)),
            pl.BlockSpec((D, J * OUT), lambda i: (0, 0)),
        ],
        out_specs=pl.BlockSpec((MM_BLOCK, J * OUT), lambda i: (i, 0)),
        out_shape=jax.ShapeDtypeStruct((N, J * OUT), jnp.float32),
    )(x, wm)

    mesh = plsc.VectorSubcoreMesh(core_axis_name="c", subcore_axis_name="s",
                                  num_cores=NC)
    partials = pl.kernel(
        _edge_body,
        out_type=jax.ShapeDtypeStruct((NC, N_PAD, OUT), jnp.float32),
        mesh=mesh,
        scratch_types=[
            pltpu.VMEM((CH,), jnp.int32),
            pltpu.VMEM((CH,), jnp.int32),
            pltpu.VMEM((J * CH, 16), jnp.float32),
            pltpu.VMEM((CH, J * OUT), jnp.float32),
            pltpu.VMEM((CH, OUT), jnp.float32),
            pltpu.VMEM_SHARED((N_PAD, OUT), jnp.float32),
            pltpu.SemaphoreType.DMA,
        ],
    )(y, src_r, dst_r, w_r)

    lin, stats = pl.pallas_call(
        _bn_stats_body,
        grid=(N // BN_BLOCK,),
        in_specs=[
            pl.BlockSpec((NC, BN_BLOCK, OUT), lambda i: (0, i, 0)),
            pl.BlockSpec((1, OUT), lambda i: (0, 0)),
        ],
        out_specs=[
            pl.BlockSpec((BN_BLOCK, OUT), lambda i: (i, 0)),
            pl.BlockSpec((2, OUT), lambda i: (0, 0)),
        ],
        out_shape=[
            jax.ShapeDtypeStruct((N, OUT), jnp.float32),
            jax.ShapeDtypeStruct((2, OUT), jnp.float32),
        ],
        scratch_shapes=[pltpu.VMEM((2, OUT), jnp.float32)],
    )(partials[:, :N, :], fc_b.reshape(1, OUT))

    out = pl.pallas_call(
        _bn_norm_body,
        grid=(N // BN_BLOCK,),
        in_specs=[
            pl.BlockSpec((BN_BLOCK, OUT), lambda i: (i, 0)),
            pl.BlockSpec((2, OUT), lambda i: (0, 0)),
            pl.BlockSpec((1, OUT), lambda i: (0, 0)),
            pl.BlockSpec((1, OUT), lambda i: (0, 0)),
        ],
        out_specs=pl.BlockSpec((BN_BLOCK, OUT), lambda i: (i, 0)),
        out_shape=jax.ShapeDtypeStruct((N, OUT), jnp.float32),
    )(lin, stats, bn_gamma.reshape(1, OUT), bn_beta.reshape(1, OUT))
    return out


# speedup vs baseline: 2.1695x; 1.2108x over previous
import jax
import jax.numpy as jnp
from jax import lax
from jax.experimental import pallas as pl
from jax.experimental.pallas import tpu as pltpu
from jax.experimental.pallas import tpu_sc as plsc

N = 10000
E = 320000
D = 128
OUT = 128
J = 2

NC = 2
NS = 16
NW = NC * NS
EPW = E // NW
CH = 40
NCH = EPW // CH
N_PAD = 10240
RPS = N_PAD // NS

MM_BLOCK = 1000
BN_BLOCK = 1000


def _mm_body(x_ref, w_ref, y_ref):
    y_ref[...] = jnp.dot(x_ref[...], w_ref[...],
                         preferred_element_type=jnp.float32)


def _edge_body(y_hbm, src_hbm, dst_hbm, w_hbm, out_hbm,
               src2, dst2, w2, rows2, msg_v, acc_sh, sem_g, sem_s):
    c = lax.axis_index("c")
    s = lax.axis_index("s")
    wid = s * NC + c

    zeros16 = jnp.zeros((16,), jnp.float32)

    def _zrow(r, carry):
        for j in range(OUT // 16):
            msg_v[r, pl.ds(16 * j, 16)] = zeros16
        return carry
    lax.fori_loop(0, CH, _zrow, 0)
    for b in range(RPS // CH):
        pltpu.sync_copy(msg_v, acc_sh.at[pl.ds(s * RPS + b * CH, CH), :])
    plsc.subcore_barrier()

    pltpu.sync_copy(src_hbm.at[wid, 0], src2.at[0])
    pltpu.sync_copy(dst_hbm.at[wid, 0], dst2.at[0])
    pltpu.sync_copy(w_hbm.at[wid, 0], w2.at[0])
    pltpu.async_copy(y_hbm.at[src2.at[0]], rows2.at[0], sem_g)

    def _pair(i, carry):
        for b in range(2):
            q = 2 * i + b
            nb = 1 - b
            d_s = pltpu.async_copy(src_hbm.at[wid, q + 1], src2.at[nb], sem_s)
            d_d = pltpu.async_copy(dst_hbm.at[wid, q + 1], dst2.at[nb], sem_s)
            d_w = pltpu.async_copy(w_hbm.at[wid, q + 1], w2.at[nb], sem_s)

            pltpu.make_async_copy(y_hbm.at[src2.at[b]], rows2.at[b],
                                  sem_g).wait()

            rv = rows2.at[b]
            wv = w2.at[b]

            def _edge(k, kcarry):
                w0 = wv[k]
                w1 = wv[CH + k]
                for j in range(OUT // 16):
                    a = rv[k, pl.ds(16 * j, 16)]
                    bb = rv[k, pl.ds(OUT + 16 * j, 16)]
                    msg_v[k, pl.ds(16 * j, 16)] = w0 * a + w1 * bb
                return kcarry
            lax.fori_loop(0, CH, _edge, 0)

            d_s.wait()
            d_d.wait()
            d_w.wait()
            pltpu.async_copy(y_hbm.at[src2.at[nb]], rows2.at[nb], sem_g)

            pltpu.sync_copy(msg_v, acc_sh.at[dst2.at[b]], add=True)
        return carry
    lax.fori_loop(0, NCH // 2, _pair, 0)
    pltpu.make_async_copy(y_hbm.at[src2.at[0]], rows2.at[0], sem_g).wait()

    plsc.subcore_barrier()
    pltpu.sync_copy(acc_sh.at[pl.ds(s * RPS, RPS), :],
                    out_hbm.at[c, pl.ds(s * RPS, RPS), :])


def _bn_stats_body(p_ref, b_ref, lin_ref, st_ref, acc_ref):
    i = pl.program_id(0)
    sm = jnp.sum(p_ref[...], axis=0) + b_ref[0]
    lin_ref[...] = sm
    blk = jnp.stack([jnp.sum(sm, axis=0), jnp.sum(sm * sm, axis=0)])

    @pl.when(i == 0)
    def _():
        acc_ref[...] = blk

    @pl.when(i > 0)
    def _():
        acc_ref[...] = acc_ref[...] + blk

    @pl.when(i == pl.num_programs(0) - 1)
    def _():
        st_ref[...] = acc_ref[...]


def _bn_norm_body(lin_ref, st_ref, g_ref, bb_ref, o_ref):
    inv_n = jnp.float32(1.0 / N)
    mean = st_ref[0] * inv_n
    var = st_ref[1] * inv_n - mean * mean
    scale = lax.rsqrt(var + jnp.float32(1e-5)) * g_ref[0]
    o_ref[...] = (lin_ref[...] - mean) * scale + bb_ref[0]


def kernel(x, W, edge_index, fc_w, fc_b, bn_gamma, bn_beta):
    wm = fc_w.reshape(OUT, J, D).transpose(2, 1, 0).reshape(D, J * OUT)
    src_r = jnp.pad(edge_index[0].reshape(NW, NCH, CH), ((0, 0), (0, 1), (0, 0)))
    dst_r = jnp.pad(edge_index[1].reshape(NW, NCH, CH), ((0, 0), (0, 1), (0, 0)))
    w_r = jnp.pad(jnp.broadcast_to(
        W.reshape(NW, NCH, CH, J).transpose(0, 1, 3, 2)
         .reshape(NW, NCH, J * CH, 1),
        (NW, NCH, J * CH, 16)).astype(jnp.float32),
        ((0, 0), (0, 1), (0, 0), (0, 0)))

    y = pl.pallas_call(
        _mm_body,
        grid=(N // MM_BLOCK,),
        in_specs=[
            pl.BlockSpec((MM_BLOCK, D), lambda i: (i, 0)),
            pl.BlockSpec((D, J * OUT), lambda i: (0, 0)),
        ],
        out_specs=pl.BlockSpec((MM_BLOCK, J * OUT), lambda i: (i, 0)),
        out_shape=jax.ShapeDtypeStruct((N, J * OUT), jnp.float32),
    )(x, wm)

    mesh = plsc.VectorSubcoreMesh(core_axis_name="c", subcore_axis_name="s",
                                  num_cores=NC)
    partials = pl.kernel(
        _edge_body,
        out_type=jax.ShapeDtypeStruct((NC, N_PAD, OUT), jnp.float32),
        mesh=mesh,
        scratch_types=[
            pltpu.VMEM((2, CH), jnp.int32),
            pltpu.VMEM((2, CH), jnp.int32),
            pltpu.VMEM((2, J * CH, 16), jnp.float32),
            pltpu.VMEM((2, CH, J * OUT), jnp.float32),
            pltpu.VMEM((CH, OUT), jnp.float32),
            pltpu.VMEM_SHARED((N_PAD, OUT), jnp.float32),
            pltpu.SemaphoreType.DMA,
            pltpu.SemaphoreType.DMA,
        ],
    )(y, src_r, dst_r, w_r)

    lin, stats = pl.pallas_call(
        _bn_stats_body,
        grid=(N // BN_BLOCK,),
        in_specs=[
            pl.BlockSpec((NC, BN_BLOCK, OUT), lambda i: (0, i, 0)),
            pl.BlockSpec((1, OUT), lambda i: (0, 0)),
        ],
        out_specs=[
            pl.BlockSpec((BN_BLOCK, OUT), lambda i: (i, 0)),
            pl.BlockSpec((2, OUT), lambda i: (0, 0)),
        ],
        out_shape=[
            jax.ShapeDtypeStruct((N, OUT), jnp.float32),
            jax.ShapeDtypeStruct((2, OUT), jnp.float32),
        ],
        scratch_shapes=[pltpu.VMEM((2, OUT), jnp.float32)],
    )(partials[:, :N, :], fc_b.reshape(1, OUT))

    out = pl.pallas_call(
        _bn_norm_body,
        grid=(N // BN_BLOCK,),
        in_specs=[
            pl.BlockSpec((BN_BLOCK, OUT), lambda i: (i, 0)),
            pl.BlockSpec((2, OUT), lambda i: (0, 0)),
            pl.BlockSpec((1, OUT), lambda i: (0, 0)),
            pl.BlockSpec((1, OUT), lambda i: (0, 0)),
        ],
        out_specs=pl.BlockSpec((BN_BLOCK, OUT), lambda i: (i, 0)),
        out_shape=jax.ShapeDtypeStruct((N, OUT), jnp.float32),
    )(lin, stats, bn_gamma.reshape(1, OUT), bn_beta.reshape(1, OUT))
    return out
